# baseline (speedup 1.0000x reference)
import jax
import jax.numpy as jnp
from jax import lax
from jax.experimental import pallas as pl
from jax.experimental.pallas import tpu as pltpu
from jax.experimental.pallas import tpu_sc as plsc

B = 16384
NF = 128
HALF = 384
NC = 2
NS = 16
L = 16
NW = NC * NS
ROWS_PER_W = B // NW
NCH = NF // L


def _sc_body(x_hbm, e_hbm, v_hbm, p0_hbm, p1_hbm, w_hbm, b_hbm, o_hbm,
             x_v, e_v, v_v, o_v, p0_v, p1_v, c0_v, c1_v, w_v, b_v,
             sem_x, sem_s):
  wid = lax.axis_index("s") * NC + lax.axis_index("c")
  base = wid * ROWS_PER_W

  cp_x = pltpu.async_copy(x_hbm.at[pl.ds(base, ROWS_PER_W), :], x_v, sem_x)
  cp = [pltpu.async_copy(e_hbm.at[pl.ds(base, ROWS_PER_W)], e_v, sem_s),
        pltpu.async_copy(v_hbm.at[pl.ds(base, ROWS_PER_W)], v_v, sem_s),
        pltpu.async_copy(p0_hbm, p0_v, sem_s),
        pltpu.async_copy(p1_hbm, p1_v, sem_s),
        pltpu.async_copy(w_hbm, w_v, sem_s),
        pltpu.async_copy(b_hbm, b_v, sem_s)]
  for c in cp:
    c.wait()

  lanes = lax.iota(jnp.int32, L)

  dnums = lax.GatherDimensionNumbers(
      offset_dims=(), collapsed_slice_dims=(0,), start_index_map=(0,))

  def permute(vec, idx):
    return lax.gather(vec, idx[:, None], dimension_numbers=dnums,
                      slice_sizes=(1,),
                      mode=lax.GatherScatterMode.PROMISE_IN_BOUNDS)

  def lane_sum(vec):
    for sh in (1, 2, 4, 8):
      vec = vec + permute(vec, lanes ^ sh)
    return vec

  for src, dst in ((p0_v, c0_v), (p1_v, c1_v)):
    for g in range(HALF // 64):
      parts = [src[pl.ds(g * 64 + L * j, L)] for j in range(64 // L)]
      total = lane_sum(parts[0] + parts[1] + parts[2] + parts[3])
      mean = total * (1.0 / 64.0)
      for j in range(64 // L):
        dst[pl.ds(g * 64 + L * j, L)] = parts[j] - mean

  bvec = b_v[pl.ds(0, L)]
  b0 = bvec[0]
  db = bvec[1] - b0
  w0s = [w_v[pl.ds(c * L, L)] for c in range(NCH)]
  dws = [w_v[pl.ds(NF + c * L, L)] - w0s[c] for c in range(NCH)]

  cp_x.wait()

  def group_body(g, carry):
    row0 = g * L
    sl = pl.ds(row0, L)
    e = e_v[sl]
    dense = jnp.zeros((L,), jnp.float32)
    for r in range(L):
      row = row0 + r
      e_r = e[r]
      acc = x_v[row, pl.ds(0, L)] * (w0s[0] + e_r * dws[0])
      for c in range(1, NCH):
        acc += x_v[row, pl.ds(c * L, L)] * (w0s[c] + e_r * dws[c])
      acc = lane_sum(acc)
      dense = jnp.where(lanes == r, acc, dense)

    v = v_v[sl]
    neg = v >= HALF
    jj = jnp.where(neg, v - HALF, v)
    sgn = jnp.where(neg, -1.0, 1.0)
    g0 = plsc.load_gather(c0_v, [jj])
    g1 = plsc.load_gather(c1_v, [jj])
    o_v[sl] = dense + b0 + e * db + sgn * (g0 + e * (g1 - g0))
    return carry

  lax.fori_loop(0, ROWS_PER_W // L, group_body, 0)

  pltpu.sync_copy(o_v, o_hbm.at[pl.ds(base, ROWS_PER_W)])


_sc_kernel = pl.kernel(
    _sc_body,
    out_type=jax.ShapeDtypeStruct((B,), jnp.float32),
    mesh=plsc.VectorSubcoreMesh(core_axis_name="c", subcore_axis_name="s"),
    compiler_params=pltpu.CompilerParams(needs_layout_passes=False),
    scratch_types=[
        pltpu.VMEM((ROWS_PER_W, NF), jnp.float32),
        pltpu.VMEM((ROWS_PER_W,), jnp.float32),
        pltpu.VMEM((ROWS_PER_W,), jnp.int32),
        pltpu.VMEM((ROWS_PER_W,), jnp.float32),
        pltpu.VMEM((HALF,), jnp.float32),
        pltpu.VMEM((HALF,), jnp.float32),
        pltpu.VMEM((HALF,), jnp.float32),
        pltpu.VMEM((HALF,), jnp.float32),
        pltpu.VMEM((2 * NF,), jnp.float32),
        pltpu.VMEM((L,), jnp.float32),
        pltpu.SemaphoreType.DMA,
        pltpu.SemaphoreType.DMA,
    ],
)


@jax.jit
def kernel(x, earliness, pst_values, pst_lengths, W, b, pst_param):
  del pst_lengths
  p0 = pst_param[:, 0]
  p1 = pst_param[:, 1]
  w_flat = W.reshape(2 * NF)
  b_pad = jnp.pad(b, (0, L - 2))
  return _sc_kernel(x, earliness, pst_values.astype(jnp.int32), p0, p1,
                    w_flat, b_pad)


# speedup vs baseline: 1.2547x; 1.2547x over previous
import jax
import jax.numpy as jnp
from jax import lax
from jax.experimental import pallas as pl
from jax.experimental.pallas import tpu as pltpu
from jax.experimental.pallas import tpu_sc as plsc

B = 16384
NF = 128
HALF = 384
NC = 2
NS = 16
L = 16
NW = NC * NS
ROWS_PER_W = B // NW
NCH = NF // L


def _sc_body(x_hbm, e_hbm, v_hbm, p0_hbm, p1_hbm, w_hbm, b_hbm, o_hbm,
             x_v, e_v, v_v, o_v, p0_v, p1_v, c0_v, c1_v, w_v, b_v,
             sem_x, sem_s):
  wid = lax.axis_index("s") * NC + lax.axis_index("c")
  base = wid * ROWS_PER_W

  cp_x = pltpu.async_copy(x_hbm.at[pl.ds(base, ROWS_PER_W), :], x_v, sem_x)
  cp = [pltpu.async_copy(e_hbm.at[pl.ds(base, ROWS_PER_W)], e_v, sem_s),
        pltpu.async_copy(v_hbm.at[pl.ds(base, ROWS_PER_W)], v_v, sem_s),
        pltpu.async_copy(p0_hbm, p0_v, sem_s),
        pltpu.async_copy(p1_hbm, p1_v, sem_s),
        pltpu.async_copy(w_hbm, w_v, sem_s),
        pltpu.async_copy(b_hbm, b_v, sem_s)]
  for c in cp:
    c.wait()

  lanes = lax.iota(jnp.int32, L)

  dnums = lax.GatherDimensionNumbers(
      offset_dims=(), collapsed_slice_dims=(0,), start_index_map=(0,))

  def permute(vec, idx):
    return lax.gather(vec, idx[:, None], dimension_numbers=dnums,
                      slice_sizes=(1,),
                      mode=lax.GatherScatterMode.PROMISE_IN_BOUNDS)

  def lane_sum(vec):
    for sh in (1, 2, 4, 8):
      vec = vec + permute(vec, lanes ^ sh)
    return vec

  for src, dst in ((p0_v, c0_v), (p1_v, c1_v)):
    for g in range(HALF // 64):
      parts = [src[pl.ds(g * 64 + L * j, L)] for j in range(64 // L)]
      total = lane_sum(parts[0] + parts[1] + parts[2] + parts[3])
      mean = total * (1.0 / 64.0)
      for j in range(64 // L):
        dst[pl.ds(g * 64 + L * j, L)] = parts[j] - mean

  bvec = b_v[pl.ds(0, L)]
  b0 = bvec[0]
  db = bvec[1] - b0
  w0s = [w_v[pl.ds(c * L, L)] for c in range(NCH)]
  dws = [w_v[pl.ds(NF + c * L, L)] - w0s[c] for c in range(NCH)]

  cp_x.wait()

  def group_body(g, carry):
    row0 = g * L
    sl = pl.ds(row0, L)
    e = e_v[sl]
    dense0 = jnp.zeros((L,), jnp.float32)
    dense1 = jnp.zeros((L,), jnp.float32)
    for r in range(L):
      row = row0 + r
      xc = x_v[row, pl.ds(0, L)]
      acc0 = xc * w0s[0]
      acc1 = xc * dws[0]
      for c in range(1, NCH):
        xc = x_v[row, pl.ds(c * L, L)]
        acc0 += xc * w0s[c]
        acc1 += xc * dws[c]
      acc0 = lane_sum(acc0)
      acc1 = lane_sum(acc1)
      dense0 = jnp.where(lanes == r, acc0, dense0)
      dense1 = jnp.where(lanes == r, acc1, dense1)

    v = v_v[sl]
    neg = v >= HALF
    jj = jnp.where(neg, v - HALF, v)
    sgn = jnp.where(neg, -1.0, 1.0)
    g0 = plsc.load_gather(c0_v, [jj])
    g1 = plsc.load_gather(c1_v, [jj])
    o_v[sl] = dense0 + b0 + e * (dense1 + db) + sgn * (g0 + e * (g1 - g0))
    return carry

  lax.fori_loop(0, ROWS_PER_W // L, group_body, 0)

  pltpu.sync_copy(o_v, o_hbm.at[pl.ds(base, ROWS_PER_W)])


_sc_kernel = pl.kernel(
    _sc_body,
    out_type=jax.ShapeDtypeStruct((B,), jnp.float32),
    mesh=plsc.VectorSubcoreMesh(core_axis_name="c", subcore_axis_name="s"),
    compiler_params=pltpu.CompilerParams(needs_layout_passes=False),
    scratch_types=[
        pltpu.VMEM((ROWS_PER_W, NF), jnp.float32),
        pltpu.VMEM((ROWS_PER_W,), jnp.float32),
        pltpu.VMEM((ROWS_PER_W,), jnp.int32),
        pltpu.VMEM((ROWS_PER_W,), jnp.float32),
        pltpu.VMEM((HALF,), jnp.float32),
        pltpu.VMEM((HALF,), jnp.float32),
        pltpu.VMEM((HALF,), jnp.float32),
        pltpu.VMEM((HALF,), jnp.float32),
        pltpu.VMEM((2 * NF,), jnp.float32),
        pltpu.VMEM((L,), jnp.float32),
        pltpu.SemaphoreType.DMA,
        pltpu.SemaphoreType.DMA,
    ],
)


@jax.jit
def kernel(x, earliness, pst_values, pst_lengths, W, b, pst_param):
  del pst_lengths
  p0 = pst_param[:, 0]
  p1 = pst_param[:, 1]
  w_flat = W.reshape(2 * NF)
  b_pad = jnp.pad(b, (0, L - 2))
  return _sc_kernel(x, earliness, pst_values.astype(jnp.int32), p0, p1,
                    w_flat, b_pad)
